# baseline (speedup 1.0000x reference)
import jax
import jax.numpy as jnp
from jax import lax
from jax.experimental import pallas as pl
from jax.experimental.pallas import tpu as pltpu
from jax.experimental.pallas import tpu_sc as plsc

_N = 10000
_E = 320000
_D = 128
_NC = 2
_NS = 16
_NW = _NC * _NS
_CH = 64
_GCH = _E // _CH
_NCH0 = _GCH // _NW
_XTRA = _GCH % _NW
_ZR = 1000

_sc_mesh = plsc.VectorSubcoreMesh(core_axis_name="c", subcore_axis_name="s")


def _sc_agg_deg_body(x_hbm, row_hbm, col_hbm, z_hbm, ones_hbm,
                     agg_hbm, deg_hbm,
                     idx_c, row_cache, rows, agg_sh,
                     sem_ic0, sem_ic1, sem_ir0, sem_ir1,
                     sem_g0, sem_g1, sem_s0, sem_s1, sem_d2, sem_d3,
                     sem_g2x, sem_g3x, sem_s2x, sem_s3x, sem_i2x, sem_i3x):
    sem_ic = (sem_ic0, sem_ic1, sem_d2, sem_d3)
    sem_ir = (sem_ir0, sem_ir1, sem_g2x, sem_g3x)
    sem_g = (sem_g0, sem_g1, sem_s2x, sem_s3x)
    sem_s = (sem_s0, sem_s1, sem_i2x, sem_i3x)
    sem_d = (sem_s0, sem_s1, sem_d2, sem_d3)
    cid = lax.axis_index("c")
    sid = lax.axis_index("s")
    wid = cid * _NS + sid

    def off(j):
        return (wid + j * _NW) * _CH

    d_ic = [None, None, None, None]
    d_ir = [None, None, None, None]
    d_g = [None, None, None, None]
    d_s = [None, None, None, None]

    def issue_idx(s, j):
        d_ic[s] = pltpu.async_copy(col_hbm.at[pl.ds(off(j), _CH)],
                                   idx_c.at[s], sem_ic[s])
        d_ir[s] = pltpu.async_copy(row_hbm.at[pl.ds(off(j), _CH)],
                                   row_cache.at[j], sem_ir[s])

    def issue_gather(s):
        d_ic[s].wait()
        d_g[s] = pltpu.async_copy(x_hbm.at[idx_c.at[s]], rows.at[s],
                                  sem_g[s])

    for _s in range(3):
        issue_idx(_s, _s)
        issue_gather(_s)

    @pl.when(sid < _N // _ZR)
    def _():
        pltpu.sync_copy(z_hbm, agg_sh.at[pl.ds(sid * _ZR, _ZR)])

    plsc.subcore_barrier()

    for j in range(_NCH0):
        s = j % 3
        d_g[s].wait()
        d_ir[s].wait()
        d_s[s] = pltpu.async_copy(rows.at[s], agg_sh.at[row_cache.at[j]],
                                  sem_s[s], add=True)
        if j + 3 < _NCH0:
            d_s[s].wait()
            issue_idx(s, j + 3)
            issue_gather(s)
    for _s in range(3):
        d_s[_s].wait()

    @pl.when(wid < _XTRA)
    def _():
        pltpu.sync_copy(col_hbm.at[pl.ds(off(_NCH0), _CH)], idx_c.at[0])
        pltpu.sync_copy(row_hbm.at[pl.ds(off(_NCH0), _CH)],
                        row_cache.at[_NCH0])
        pltpu.sync_copy(x_hbm.at[idx_c.at[0]], rows.at[0])
        pltpu.sync_copy(rows.at[0], agg_sh.at[row_cache.at[_NCH0]], add=True)

    plsc.subcore_barrier()

    @pl.when(sid < _N // _ZR)
    def _():
        pltpu.sync_copy(agg_sh.at[pl.ds(sid * _ZR, _ZR)],
                        agg_hbm.at[cid, pl.ds(sid * _ZR, _ZR)])

    plsc.subcore_barrier()

    @pl.when(sid < _N // _ZR)
    def _():
        pltpu.sync_copy(z_hbm, agg_sh.at[pl.ds(sid * _ZR, _ZR)])

    pltpu.sync_copy(ones_hbm, rows.at[0])
    plsc.subcore_barrier()

    d_d = [None, None, None, None]
    for j in range(_NCH0):
        s = j % 3
        if d_d[s] is not None:
            d_d[s].wait()
        d_d[s] = pltpu.async_copy(rows.at[0], agg_sh.at[row_cache.at[j]],
                                  sem_d[s], add=True)
    for s in range(3):
        d_d[s].wait()

    @pl.when(wid < _XTRA)
    def _():
        pltpu.sync_copy(rows.at[0], agg_sh.at[row_cache.at[_NCH0]], add=True)

    plsc.subcore_barrier()

    @pl.when(sid < _N // _ZR)
    def _():
        pltpu.sync_copy(agg_sh.at[pl.ds(sid * _ZR, _ZR)],
                        deg_hbm.at[cid, pl.ds(sid * _ZR, _ZR)])


def _sc_agg_body(x_hbm, row_hbm, col_hbm, z_hbm, agg_hbm,
                 idx_c, idx_r, rows, agg_sh,
                 sem_ic0, sem_ic1, sem_ir0, sem_ir1,
                 sem_g0, sem_g1, sem_s0, sem_s1,
                 sem_g2x, sem_g3x, sem_s2x, sem_s3x, sem_i2x, sem_i3x,
                 sem_c2x, sem_c3x):
    sem_ic = (sem_ic0, sem_ic1, sem_c2x, sem_c3x)
    sem_ir = (sem_ir0, sem_ir1, sem_g2x, sem_g3x)
    sem_g = (sem_g0, sem_g1, sem_s2x, sem_s3x)
    sem_s = (sem_s0, sem_s1, sem_i2x, sem_i3x)
    cid = lax.axis_index("c")
    sid = lax.axis_index("s")
    wid = cid * _NS + sid

    def off(j):
        return (wid + j * _NW) * _CH

    d_ic = [None, None, None, None]
    d_ir = [None, None, None, None]
    d_g = [None, None, None, None]
    d_s = [None, None, None, None]

    def issue_idx(s, j):
        d_ic[s] = pltpu.async_copy(col_hbm.at[pl.ds(off(j), _CH)],
                                   idx_c.at[s], sem_ic[s])
        d_ir[s] = pltpu.async_copy(row_hbm.at[pl.ds(off(j), _CH)],
                                   idx_r.at[s], sem_ir[s])

    def issue_gather(s):
        d_ic[s].wait()
        d_g[s] = pltpu.async_copy(x_hbm.at[idx_c.at[s]], rows.at[s],
                                  sem_g[s])

    for _s in range(3):
        issue_idx(_s, _s)
        issue_gather(_s)

    @pl.when(sid < _N // _ZR)
    def _():
        pltpu.sync_copy(z_hbm, agg_sh.at[pl.ds(sid * _ZR, _ZR)])

    plsc.subcore_barrier()

    for j in range(_NCH0):
        s = j % 3
        d_g[s].wait()
        d_ir[s].wait()
        d_s[s] = pltpu.async_copy(rows.at[s], agg_sh.at[idx_r.at[s]],
                                  sem_s[s], add=True)
        if j + 3 < _NCH0:
            d_s[s].wait()
            issue_idx(s, j + 3)
            issue_gather(s)
    for _s in range(3):
        d_s[_s].wait()

    @pl.when(wid < _XTRA)
    def _():
        pltpu.sync_copy(col_hbm.at[pl.ds(off(_NCH0), _CH)], idx_c.at[0])
        pltpu.sync_copy(row_hbm.at[pl.ds(off(_NCH0), _CH)], idx_r.at[0])
        pltpu.sync_copy(x_hbm.at[idx_c.at[0]], rows.at[0])
        pltpu.sync_copy(rows.at[0], agg_sh.at[idx_r.at[0]], add=True)

    plsc.subcore_barrier()

    @pl.when(sid < _N // _ZR)
    def _():
        pltpu.sync_copy(agg_sh.at[pl.ds(sid * _ZR, _ZR)],
                        agg_hbm.at[cid, pl.ds(sid * _ZR, _ZR)])


_sc_agg_deg = pl.kernel(
    _sc_agg_deg_body,
    out_type=(jax.ShapeDtypeStruct((_NC, _N, _D), jnp.float32),
              jax.ShapeDtypeStruct((_NC, _N, _D), jnp.float32)),
    mesh=_sc_mesh,
    scratch_types=[
        pltpu.VMEM((3, _CH), jnp.int32),
        pltpu.VMEM((_NCH0 + 1, _CH), jnp.int32),
        pltpu.VMEM((3, _CH, _D), jnp.float32),
        pltpu.VMEM_SHARED((_N, _D), jnp.float32),
    ] + [pltpu.SemaphoreType.DMA] * 16,
)

_sc_agg = pl.kernel(
    _sc_agg_body,
    out_type=jax.ShapeDtypeStruct((_NC, _N, _D), jnp.float32),
    mesh=_sc_mesh,
    scratch_types=[
        pltpu.VMEM((3, _CH), jnp.int32),
        pltpu.VMEM((3, _CH), jnp.int32),
        pltpu.VMEM((3, _CH, _D), jnp.float32),
        pltpu.VMEM_SHARED((_N, _D), jnp.float32),
    ] + [pltpu.SemaphoreType.DMA] * 16,
)


def _tc1_body(x_ref, a0_ref, a1_ref, d0_ref, d1_ref, wt_ref, b_ref, g_ref,
              be_ref, o_ref, dinv_ref):
    deg = jnp.maximum(d0_ref[:, 0:1] + d1_ref[:, 0:1], 1.0)
    dinv = 1.0 / deg
    h = x_ref[...] + (a0_ref[...] + a1_ref[...]) * dinv
    out = jnp.dot(h, wt_ref[...], preferred_element_type=jnp.float32,
                  precision=lax.Precision.HIGHEST) + b_ref[...]
    mu = jnp.mean(out, axis=1, keepdims=True)
    var = jnp.mean(jnp.square(out - mu), axis=1, keepdims=True)
    out = (out - mu) * lax.rsqrt(var + 1e-5) * g_ref[...] + be_ref[...]
    o_ref[...] = out * (1.0 / (1.0 + jnp.exp(-out)))
    dinv_ref[...] = jnp.broadcast_to(dinv, (_N, 8))


def _tc2_body(x_ref, a0_ref, a1_ref, dinv_ref, wt_ref, b_ref, g_ref, be_ref,
              o_ref):
    h = x_ref[...] + (a0_ref[...] + a1_ref[...]) * dinv_ref[:, 0:1]
    out = jnp.dot(h, wt_ref[...], preferred_element_type=jnp.float32,
                  precision=lax.Precision.HIGHEST) + b_ref[...]
    mu = jnp.mean(out, axis=1, keepdims=True)
    var = jnp.mean(jnp.square(out - mu), axis=1, keepdims=True)
    out = (out - mu) * lax.rsqrt(var + 1e-5) * g_ref[...] + be_ref[...]
    o_ref[...] = out * (1.0 / (1.0 + jnp.exp(-out)))


_tc_pass1 = pl.pallas_call(
    _tc1_body,
    out_shape=(jax.ShapeDtypeStruct((_N, _D), jnp.float32),
               jax.ShapeDtypeStruct((_N, 8), jnp.float32)),
)

_tc_pass2 = pl.pallas_call(
    _tc2_body,
    out_shape=jax.ShapeDtypeStruct((_N, _D), jnp.float32),
)


def kernel(x, edge_index, W1, b1, g1, be1, W2, b2, g2, be2):
    row = edge_index[0]
    col = edge_index[1]
    zeros_r = jnp.zeros((_ZR, _D), jnp.float32)
    ones_r = jnp.ones((_CH, _D), jnp.float32)

    aggp, degp = _sc_agg_deg(x, row, col, zeros_r, ones_r)
    h, dinv = _tc_pass1(x, aggp[0], aggp[1], degp[0], degp[1], W1.T,
                        b1.reshape(1, _D), g1.reshape(1, _D),
                        be1.reshape(1, _D))
    aggp2 = _sc_agg(h, row, col, zeros_r)
    out = _tc_pass2(h, aggp2[0], aggp2[1], dinv, W2.T,
                    b2.reshape(1, _D), g2.reshape(1, _D), be2.reshape(1, _D))
    return out


# speedup vs baseline: 1.2198x; 1.2198x over previous
import jax
import jax.numpy as jnp
from jax import lax
from jax.experimental import pallas as pl
from jax.experimental.pallas import tpu as pltpu
from jax.experimental.pallas import tpu_sc as plsc

_N = 10000
_E = 320000
_D = 128
_NC = 2
_NS = 16
_NW = _NC * _NS
_CH = 128
_GCH = _E // _CH
_NCH0 = _GCH // _NW
_XTRA = _GCH % _NW
_ZR = 1000

_sc_mesh = plsc.VectorSubcoreMesh(core_axis_name="c", subcore_axis_name="s")


def _sc_agg_deg_body(x_hbm, row_hbm, col_hbm, z_hbm, ones_hbm,
                     agg_hbm, deg_hbm,
                     idx_c, row_cache, rows, agg_sh,
                     sem_ic0, sem_ic1, sem_ir0, sem_ir1,
                     sem_g0, sem_g1, sem_s0, sem_s1, sem_d2, sem_d3):
    sem_ic = (sem_ic0, sem_ic1)
    sem_ir = (sem_ir0, sem_ir1)
    sem_g = (sem_g0, sem_g1)
    sem_s = (sem_s0, sem_s1)
    sem_d = (sem_s0, sem_s1, sem_d2, sem_d3)
    cid = lax.axis_index("c")
    sid = lax.axis_index("s")
    wid = cid * _NS + sid

    def off(j):
        return (wid + j * _NW) * _CH

    d_ic = [None, None]
    d_ir = [None, None]
    d_g = [None, None]
    d_s = [None, None]

    def issue_idx(s, j):
        d_ic[s] = pltpu.async_copy(col_hbm.at[pl.ds(off(j), _CH)],
                                   idx_c.at[s], sem_ic[s])
        d_ir[s] = pltpu.async_copy(row_hbm.at[pl.ds(off(j), _CH)],
                                   row_cache.at[j], sem_ir[s])

    def issue_gather(s):
        d_ic[s].wait()
        d_g[s] = pltpu.async_copy(x_hbm.at[idx_c.at[s]], rows.at[s],
                                  sem_g[s])

    issue_idx(0, 0)
    issue_idx(1, 1)
    issue_gather(0)
    issue_gather(1)

    @pl.when(sid < _N // _ZR)
    def _():
        pltpu.sync_copy(z_hbm, agg_sh.at[pl.ds(sid * _ZR, _ZR)])

    plsc.subcore_barrier()

    for j in range(_NCH0):
        s = j & 1
        d_g[s].wait()
        d_ir[s].wait()
        d_s[s] = pltpu.async_copy(rows.at[s], agg_sh.at[row_cache.at[j]],
                                  sem_s[s], add=True)
        if j + 2 < _NCH0:
            issue_idx(s, j + 2)
            d_s[s].wait()
            issue_gather(s)
    d_s[0].wait()
    d_s[1].wait()

    @pl.when(wid < _XTRA)
    def _():
        pltpu.sync_copy(col_hbm.at[pl.ds(off(_NCH0), _CH)], idx_c.at[0])
        pltpu.sync_copy(row_hbm.at[pl.ds(off(_NCH0), _CH)],
                        row_cache.at[_NCH0])
        pltpu.sync_copy(x_hbm.at[idx_c.at[0]], rows.at[0])
        pltpu.sync_copy(rows.at[0], agg_sh.at[row_cache.at[_NCH0]], add=True)

    plsc.subcore_barrier()

    @pl.when(sid < _N // _ZR)
    def _():
        pltpu.sync_copy(agg_sh.at[pl.ds(sid * _ZR, _ZR)],
                        agg_hbm.at[cid, pl.ds(sid * _ZR, _ZR)])

    plsc.subcore_barrier()

    @pl.when(sid < _N // _ZR)
    def _():
        pltpu.sync_copy(z_hbm, agg_sh.at[pl.ds(sid * _ZR, _ZR)])

    pltpu.sync_copy(ones_hbm, rows.at[0])
    plsc.subcore_barrier()

    d_d = [None, None, None, None]
    for j in range(_NCH0):
        s = j & 3
        if d_d[s] is not None:
            d_d[s].wait()
        d_d[s] = pltpu.async_copy(rows.at[0], agg_sh.at[row_cache.at[j]],
                                  sem_d[s], add=True)
    for s in range(4):
        d_d[s].wait()

    @pl.when(wid < _XTRA)
    def _():
        pltpu.sync_copy(rows.at[0], agg_sh.at[row_cache.at[_NCH0]], add=True)

    plsc.subcore_barrier()

    @pl.when(sid < _N // _ZR)
    def _():
        pltpu.sync_copy(agg_sh.at[pl.ds(sid * _ZR, _ZR)],
                        deg_hbm.at[cid, pl.ds(sid * _ZR, _ZR)])


def _sc_agg_body(x_hbm, row_hbm, col_hbm, z_hbm, agg_hbm,
                 idx_c, idx_r, rows, agg_sh,
                 sem_ic0, sem_ic1, sem_ir0, sem_ir1,
                 sem_g0, sem_g1, sem_s0, sem_s1, sem_ir2, sem_ir3):
    sem_ic = (sem_ic0, sem_ic1)
    sem_ir = (sem_ir0, sem_ir1, sem_ir2, sem_ir3)
    sem_g = (sem_g0, sem_g1)
    sem_s = (sem_s0, sem_s1)
    cid = lax.axis_index("c")
    sid = lax.axis_index("s")
    wid = cid * _NS + sid

    def off(j):
        return (wid + j * _NW) * _CH

    d_ic = [None, None]
    d_ir = [None, None, None, None]
    d_g = [None, None]
    d_s = [None, None]

    def issue_idx(s, j):
        r = j & 3
        d_ic[s] = pltpu.async_copy(col_hbm.at[pl.ds(off(j), _CH)],
                                   idx_c.at[s], sem_ic[s])
        d_ir[r] = pltpu.async_copy(row_hbm.at[pl.ds(off(j), _CH)],
                                   idx_r.at[r], sem_ir[r])

    def issue_gather(s):
        d_ic[s].wait()
        d_g[s] = pltpu.async_copy(x_hbm.at[idx_c.at[s]], rows.at[s],
                                  sem_g[s])

    issue_idx(0, 0)
    issue_idx(1, 1)
    issue_gather(0)
    issue_gather(1)

    @pl.when(sid < _N // _ZR)
    def _():
        pltpu.sync_copy(z_hbm, agg_sh.at[pl.ds(sid * _ZR, _ZR)])

    plsc.subcore_barrier()

    for j in range(_NCH0):
        s = j & 1
        r = j & 3
        d_g[s].wait()
        d_ir[r].wait()
        d_s[s] = pltpu.async_copy(rows.at[s], agg_sh.at[idx_r.at[r]],
                                  sem_s[s], add=True)
        if j + 2 < _NCH0:
            issue_idx(s, j + 2)
            d_s[s].wait()
            issue_gather(s)
    d_s[0].wait()
    d_s[1].wait()

    @pl.when(wid < _XTRA)
    def _():
        pltpu.sync_copy(col_hbm.at[pl.ds(off(_NCH0), _CH)], idx_c.at[0])
        pltpu.sync_copy(row_hbm.at[pl.ds(off(_NCH0), _CH)], idx_r.at[0])
        pltpu.sync_copy(x_hbm.at[idx_c.at[0]], rows.at[0])
        pltpu.sync_copy(rows.at[0], agg_sh.at[idx_r.at[0]], add=True)

    plsc.subcore_barrier()

    @pl.when(sid < _N // _ZR)
    def _():
        pltpu.sync_copy(agg_sh.at[pl.ds(sid * _ZR, _ZR)],
                        agg_hbm.at[cid, pl.ds(sid * _ZR, _ZR)])


_sc_agg_deg = pl.kernel(
    _sc_agg_deg_body,
    out_type=(jax.ShapeDtypeStruct((_NC, _N, _D), jnp.float32),
              jax.ShapeDtypeStruct((_NC, _N, _D), jnp.float32)),
    mesh=_sc_mesh,
    scratch_types=[
        pltpu.VMEM((2, _CH), jnp.int32),
        pltpu.VMEM((_NCH0 + 1, _CH), jnp.int32),
        pltpu.VMEM((2, _CH, _D), jnp.float32),
        pltpu.VMEM_SHARED((_N, _D), jnp.float32),
    ] + [pltpu.SemaphoreType.DMA] * 10,
)

_sc_agg = pl.kernel(
    _sc_agg_body,
    out_type=jax.ShapeDtypeStruct((_NC, _N, _D), jnp.float32),
    mesh=_sc_mesh,
    scratch_types=[
        pltpu.VMEM((2, _CH), jnp.int32),
        pltpu.VMEM((4, _CH), jnp.int32),
        pltpu.VMEM((2, _CH, _D), jnp.float32),
        pltpu.VMEM_SHARED((_N, _D), jnp.float32),
    ] + [pltpu.SemaphoreType.DMA] * 10,
)


def _tc1_body(x_ref, a0_ref, a1_ref, d0_ref, d1_ref, wt_ref, b_ref, g_ref,
              be_ref, o_ref, dinv_ref):
    deg = jnp.maximum(d0_ref[:, 0:1] + d1_ref[:, 0:1], 1.0)
    dinv = 1.0 / deg
    h = x_ref[...] + (a0_ref[...] + a1_ref[...]) * dinv
    out = jnp.dot(h, wt_ref[...], preferred_element_type=jnp.float32,
                  precision=lax.Precision.HIGHEST) + b_ref[...]
    mu = jnp.mean(out, axis=1, keepdims=True)
    var = jnp.mean(jnp.square(out - mu), axis=1, keepdims=True)
    out = (out - mu) * lax.rsqrt(var + 1e-5) * g_ref[...] + be_ref[...]
    o_ref[...] = out * (1.0 / (1.0 + jnp.exp(-out)))
    dinv_ref[...] = jnp.broadcast_to(dinv, (_N, 8))


def _tc2_body(x_ref, a0_ref, a1_ref, dinv_ref, wt_ref, b_ref, g_ref, be_ref,
              o_ref):
    h = x_ref[...] + (a0_ref[...] + a1_ref[...]) * dinv_ref[:, 0:1]
    out = jnp.dot(h, wt_ref[...], preferred_element_type=jnp.float32,
                  precision=lax.Precision.HIGHEST) + b_ref[...]
    mu = jnp.mean(out, axis=1, keepdims=True)
    var = jnp.mean(jnp.square(out - mu), axis=1, keepdims=True)
    out = (out - mu) * lax.rsqrt(var + 1e-5) * g_ref[...] + be_ref[...]
    o_ref[...] = out * (1.0 / (1.0 + jnp.exp(-out)))


_tc_pass1 = pl.pallas_call(
    _tc1_body,
    out_shape=(jax.ShapeDtypeStruct((_N, _D), jnp.float32),
               jax.ShapeDtypeStruct((_N, 8), jnp.float32)),
)

_tc_pass2 = pl.pallas_call(
    _tc2_body,
    out_shape=jax.ShapeDtypeStruct((_N, _D), jnp.float32),
)


def kernel(x, edge_index, W1, b1, g1, be1, W2, b2, g2, be2):
    row = edge_index[0]
    col = edge_index[1]
    zeros_r = jnp.zeros((_ZR, _D), jnp.float32)
    ones_r = jnp.ones((_CH, _D), jnp.float32)

    aggp, degp = _sc_agg_deg(x, row, col, zeros_r, ones_r)
    h, dinv = _tc_pass1(x, aggp[0], aggp[1], degp[0], degp[1], W1.T,
                        b1.reshape(1, _D), g1.reshape(1, _D),
                        be1.reshape(1, _D))
    aggp2 = _sc_agg(h, row, col, zeros_r)
    out = _tc_pass2(h, aggp2[0], aggp2[1], dinv, W2.T,
                    b2.reshape(1, _D), g2.reshape(1, _D), be2.reshape(1, _D))
    return out
